# Initial kernel scaffold; baseline (speedup 1.0000x reference)
import functools

import jax
import jax.numpy as jnp
from jax import lax
from jax.experimental import pallas as pl
from jax.experimental.pallas import tpu as pltpu
from jax.experimental.pallas import tpu_sc as plsc

CHUNK = 128
NC = 2
NS = 16
NW = NC * NS


def _sc_segment_sums(x, i_ch, j_ch, ea_ch, n_acc):
    n, d = x.shape
    c_total = i_ch.shape[0]
    de = ea_ch.shape[2]
    rows_pt = n_acc // NS

    base_cnt = c_total // NW
    n_extra = c_total - base_cnt * NW

    zx = jnp.zeros((rows_pt, d), jnp.float32)
    zea = jnp.zeros((rows_pt, de), jnp.float32)
    ones = jnp.ones((CHUNK, de), jnp.float32)

    mesh = plsc.VectorSubcoreMesh(core_axis_name="c", subcore_axis_name="s")

    @functools.partial(
        pl.kernel,
        out_type=[
            jax.ShapeDtypeStruct((NC, n_acc, d), jnp.float32),
            jax.ShapeDtypeStruct((NC, n_acc, de), jnp.float32),
            jax.ShapeDtypeStruct((NC, n_acc, de), jnp.float32),
        ],
        mesh=mesh,
        scratch_types=[
            pltpu.VMEM_SHARED((n_acc, d), jnp.float32),
            pltpu.VMEM_SHARED((n_acc, de), jnp.float32),
            pltpu.VMEM_SHARED((n_acc, de), jnp.float32),
            pltpu.VMEM((CHUNK,), jnp.int32),
            pltpu.VMEM((CHUNK,), jnp.int32),
            pltpu.VMEM((CHUNK, d), jnp.float32),
            pltpu.VMEM((CHUNK, de), jnp.float32),
            pltpu.VMEM((CHUNK, de), jnp.float32),
            pltpu.SemaphoreType.DMA,
        ],
    )
    def sc_kernel(x_hbm, i_hbm, j_hbm, ea_hbm, zx_hbm, zea_hbm, ones_hbm,
                  outx_hbm, outea_hbm, outd_hbm,
                  accx, accea, accd, i_v, j_v, rows_v, ea_v, ones_v, sem):
        cid = lax.axis_index("c")
        sid = lax.axis_index("s")
        wid = sid * NC + cid

        pltpu.sync_copy(zx_hbm, accx.at[pl.ds(sid * rows_pt, rows_pt)])
        pltpu.sync_copy(zea_hbm, accea.at[pl.ds(sid * rows_pt, rows_pt)])
        pltpu.sync_copy(zea_hbm, accd.at[pl.ds(sid * rows_pt, rows_pt)])
        pltpu.sync_copy(ones_hbm, ones_v)
        plsc.subcore_barrier()

        start = wid * base_cnt + jnp.minimum(wid, n_extra)
        cnt = base_cnt + jnp.where(wid < n_extra, 1, 0)

        def body(k, carry):
            row = start + k
            pltpu.sync_copy(i_hbm.at[row], i_v)
            pltpu.sync_copy(j_hbm.at[row], j_v)
            pltpu.async_copy(x_hbm.at[j_v], rows_v, sem).wait()
            pltpu.sync_copy(ea_hbm.at[row], ea_v)
            pltpu.sync_copy(rows_v, accx.at[i_v], add=True)
            pltpu.sync_copy(ea_v, accea.at[i_v], add=True)
            pltpu.sync_copy(ones_v, accd.at[i_v], add=True)
            return carry

        lax.fori_loop(0, cnt, body, 0)
        plsc.subcore_barrier()

        sl = pl.ds(sid * rows_pt, rows_pt)
        pltpu.sync_copy(accx.at[sl], outx_hbm.at[cid, sl])
        pltpu.sync_copy(accea.at[sl], outea_hbm.at[cid, sl])
        pltpu.sync_copy(accd.at[sl], outd_hbm.at[cid, sl])

    return sc_kernel(x, i_ch, j_ch, ea_ch, zx, zea, ones)


def _tc_dense(x, px, pea, pd, eps, W1, b1, W2, b2, We, be, gamma, beta):
    n, d = x.shape
    de = We.shape[0]
    blk = 1000
    grid = n // blk

    epsv = jnp.reshape(1.0 + eps, (1, 1)).astype(jnp.float32)
    b1v = b1.reshape(1, d)
    b2v = b2.reshape(1, d)
    bev = be.reshape(1, d)
    gammav = gamma.reshape(1, d)
    betav = beta.reshape(1, d)

    def body(x_ref, px_ref, pea_ref, pd_ref, eps_ref, w1_ref, b1_ref,
             w2_ref, b2_ref, we_ref, be_ref, g_ref, bt_ref, o_ref):
        agg = px_ref[0] + px_ref[1]
        aea = pea_ref[0] + pea_ref[1]
        deg = pd_ref[0, :, 0:1] + pd_ref[1, :, 0:1]
        h = (eps_ref[0, 0] * x_ref[...] + agg
             + jnp.dot(aea, we_ref[...], preferred_element_type=jnp.float32)
             + deg * be_ref[...])
        h = jnp.maximum(
            jnp.dot(h, w1_ref[...], preferred_element_type=jnp.float32)
            + b1_ref[...], 0.0)
        h = jnp.dot(h, w2_ref[...], preferred_element_type=jnp.float32) + b2_ref[...]
        mu = jnp.mean(h, axis=-1, keepdims=True)
        hc = h - mu
        var = jnp.mean(hc * hc, axis=-1, keepdims=True)
        o_ref[...] = hc * lax.rsqrt(var + 1e-5) * g_ref[...] + bt_ref[...]

    full = lambda i: (0, 0)
    return pl.pallas_call(
        body,
        grid=(grid,),
        in_specs=[
            pl.BlockSpec((blk, d), lambda i: (i, 0)),
            pl.BlockSpec((NC, blk, d), lambda i: (0, i, 0)),
            pl.BlockSpec((NC, blk, de), lambda i: (0, i, 0)),
            pl.BlockSpec((NC, blk, de), lambda i: (0, i, 0)),
            pl.BlockSpec((1, 1), full),
            pl.BlockSpec((d, d), full),
            pl.BlockSpec((1, d), full),
            pl.BlockSpec((d, d), full),
            pl.BlockSpec((1, d), full),
            pl.BlockSpec((de, d), full),
            pl.BlockSpec((1, d), full),
            pl.BlockSpec((1, d), full),
            pl.BlockSpec((1, d), full),
        ],
        out_specs=pl.BlockSpec((blk, d), lambda i: (i, 0)),
        out_shape=jax.ShapeDtypeStruct((n, d), jnp.float32),
    )(x, px, pea, pd, epsv, W1, b1v, W2, b2v, We, bev, gammav, betav)


def kernel(x, edge_index, edge_attr, eps, W1, b1, W2, b2, We, be, gamma, beta):
    n, d = x.shape
    e, de = edge_attr.shape
    assert e % CHUNK == 0

    ei = edge_index.astype(jnp.int32)
    c_total = e // CHUNK
    i_ch = ei[0].reshape(c_total, CHUNK)
    j_ch = ei[1].reshape(c_total, CHUNK)
    ea_ch = edge_attr.reshape(c_total, CHUNK, de)

    n_acc = -(-n // (NS * CHUNK)) * (NS * CHUNK)

    px, pea, pd = _sc_segment_sums(x, i_ch, j_ch, ea_ch, n_acc)
    return _tc_dense(x, px, pea, pd, eps, W1, b1, W2, b2, We, be, gamma, beta)


# speedup vs baseline: 3.0202x; 3.0202x over previous
import functools

import jax
import jax.numpy as jnp
from jax import lax
from jax.experimental import pallas as pl
from jax.experimental.pallas import tpu as pltpu
from jax.experimental.pallas import tpu_sc as plsc

CHUNK = 64
NC = 2
NS = 16
NW = NC * NS


def _sc_segment_sums(x, i_ch, j_ch, ea_ch, n_acc):
    n, d = x.shape
    c_total = i_ch.shape[0]
    de = ea_ch.shape[2]
    rows_pt = n_acc // NS

    base_cnt = c_total // NW
    n_extra = c_total - base_cnt * NW

    zx = jnp.zeros((rows_pt, d), jnp.float32)
    zea = jnp.zeros((rows_pt, de), jnp.float32)
    ones = jnp.ones((CHUNK, de), jnp.float32)

    mesh = plsc.VectorSubcoreMesh(core_axis_name="c", subcore_axis_name="s")

    @functools.partial(
        pl.kernel,
        out_type=[
            jax.ShapeDtypeStruct((NC, n_acc, d), jnp.float32),
            jax.ShapeDtypeStruct((NC, n_acc, de), jnp.float32),
            jax.ShapeDtypeStruct((NC, n_acc, de), jnp.float32),
        ],
        mesh=mesh,
        compiler_params=pltpu.CompilerParams(use_tc_tiling_on_sc=False),
        scratch_types=[
            pltpu.VMEM_SHARED((n_acc, d), jnp.float32),
            pltpu.VMEM_SHARED((n_acc, de), jnp.float32),
            pltpu.VMEM_SHARED((n_acc, de), jnp.float32),
            pltpu.VMEM((CHUNK,), jnp.int32),
            pltpu.VMEM((CHUNK,), jnp.int32),
            pltpu.VMEM((CHUNK, d), jnp.float32),
            pltpu.VMEM((CHUNK, de), jnp.float32),
            pltpu.VMEM((CHUNK, de), jnp.float32),
            pltpu.SemaphoreType.DMA,
        ],
    )
    def sc_kernel(x_hbm, i_hbm, j_hbm, ea_hbm, zx_hbm, zea_hbm, ones_hbm,
                  outx_hbm, outea_hbm, outd_hbm,
                  accx, accea, accd, i_v, j_v, rows_v, ea_v, ones_v, sem):
        cid = lax.axis_index("c")
        sid = lax.axis_index("s")
        wid = sid * NC + cid

        pltpu.sync_copy(zx_hbm, accx.at[pl.ds(sid * rows_pt, rows_pt)])
        pltpu.sync_copy(zea_hbm, accea.at[pl.ds(sid * rows_pt, rows_pt)])
        pltpu.sync_copy(zea_hbm, accd.at[pl.ds(sid * rows_pt, rows_pt)])
        pltpu.sync_copy(ones_hbm, ones_v)
        plsc.subcore_barrier()

        start = wid * base_cnt + jnp.minimum(wid, n_extra)
        cnt = base_cnt + jnp.where(wid < n_extra, 1, 0)

        def body(k, carry):
            row = start + k
            pltpu.sync_copy(i_hbm.at[row], i_v)
            pltpu.sync_copy(j_hbm.at[row], j_v)
            pltpu.async_copy(x_hbm.at[j_v], rows_v, sem).wait()
            pltpu.sync_copy(ea_hbm.at[row], ea_v)
            pltpu.sync_copy(rows_v, accx.at[i_v], add=True)
            pltpu.sync_copy(ea_v, accea.at[i_v], add=True)
            pltpu.sync_copy(ones_v, accd.at[i_v], add=True)
            return carry

        lax.fori_loop(0, cnt, body, 0)
        plsc.subcore_barrier()

        sl = pl.ds(sid * rows_pt, rows_pt)
        pltpu.sync_copy(accx.at[sl], outx_hbm.at[cid, sl])
        pltpu.sync_copy(accea.at[sl], outea_hbm.at[cid, sl])
        pltpu.sync_copy(accd.at[sl], outd_hbm.at[cid, sl])

    return sc_kernel(x, i_ch, j_ch, ea_ch, zx, zea, ones)


def _tc_dense(x, px, pea, pd, eps, W1, b1, W2, b2, We, be, gamma, beta):
    n, d = x.shape
    de = We.shape[0]
    blk = 1000
    grid = n // blk

    epsv = jnp.reshape(1.0 + eps, (1, 1)).astype(jnp.float32)
    b1v = b1.reshape(1, d)
    b2v = b2.reshape(1, d)
    bev = be.reshape(1, d)
    gammav = gamma.reshape(1, d)
    betav = beta.reshape(1, d)

    def body(x_ref, px_ref, pea_ref, pd_ref, eps_ref, w1_ref, b1_ref,
             w2_ref, b2_ref, we_ref, be_ref, g_ref, bt_ref, o_ref):
        agg = px_ref[0] + px_ref[1]
        aea = pea_ref[0] + pea_ref[1]
        deg = pd_ref[0, :, 0:1] + pd_ref[1, :, 0:1]
        h = (eps_ref[0, 0] * x_ref[...] + agg
             + jnp.dot(aea, we_ref[...], preferred_element_type=jnp.float32)
             + deg * be_ref[...])
        h = jnp.maximum(
            jnp.dot(h, w1_ref[...], preferred_element_type=jnp.float32)
            + b1_ref[...], 0.0)
        h = jnp.dot(h, w2_ref[...], preferred_element_type=jnp.float32) + b2_ref[...]
        mu = jnp.mean(h, axis=-1, keepdims=True)
        hc = h - mu
        var = jnp.mean(hc * hc, axis=-1, keepdims=True)
        o_ref[...] = hc * lax.rsqrt(var + 1e-5) * g_ref[...] + bt_ref[...]

    full = lambda i: (0, 0)
    return pl.pallas_call(
        body,
        grid=(grid,),
        in_specs=[
            pl.BlockSpec((blk, d), lambda i: (i, 0)),
            pl.BlockSpec((NC, blk, d), lambda i: (0, i, 0)),
            pl.BlockSpec((NC, blk, de), lambda i: (0, i, 0)),
            pl.BlockSpec((NC, blk, de), lambda i: (0, i, 0)),
            pl.BlockSpec((1, 1), full),
            pl.BlockSpec((d, d), full),
            pl.BlockSpec((1, d), full),
            pl.BlockSpec((d, d), full),
            pl.BlockSpec((1, d), full),
            pl.BlockSpec((de, d), full),
            pl.BlockSpec((1, d), full),
            pl.BlockSpec((1, d), full),
            pl.BlockSpec((1, d), full),
        ],
        out_specs=pl.BlockSpec((blk, d), lambda i: (i, 0)),
        out_shape=jax.ShapeDtypeStruct((n, d), jnp.float32),
    )(x, px, pea, pd, epsv, W1, b1v, W2, b2v, We, bev, gammav, betav)


def kernel(x, edge_index, edge_attr, eps, W1, b1, W2, b2, We, be, gamma, beta):
    n, d = x.shape
    e, de = edge_attr.shape
    assert e % CHUNK == 0

    ei = edge_index.astype(jnp.int32)
    c_total = e // CHUNK
    i_ch = ei[0].reshape(c_total, CHUNK)
    j_ch = ei[1].reshape(c_total, CHUNK)
    ea_ch = edge_attr.reshape(c_total, CHUNK, de)

    n_acc = -(-n // (NS * 8)) * (NS * 8)

    px, pea, pd = _sc_segment_sums(x, i_ch, j_ch, ea_ch, n_acc)
    return _tc_dense(x, px, pea, pd, eps, W1, b1, W2, b2, We, be, gamma, beta)


# speedup vs baseline: 4.9829x; 1.6499x over previous
import functools

import jax
import jax.numpy as jnp
from jax import lax
from jax.experimental import pallas as pl
from jax.experimental.pallas import tpu as pltpu
from jax.experimental.pallas import tpu_sc as plsc

CHUNK = 80
NC = 2
NS = 16
NW = NC * NS


def _sc_segment_sums(xb, ei, ea, n_acc):
    n, d = xb.shape
    e = ei.shape[1]
    de = ea.shape[1]
    rows_pt = n_acc // NS
    assert e % (CHUNK * NW) == 0
    cpw = e // (CHUNK * NW)
    assert cpw >= 3 and cpw % 2 == 1

    zx = jnp.zeros((rows_pt, d), jnp.float32)
    zea = jnp.zeros((rows_pt, de), jnp.float32)

    mesh = plsc.VectorSubcoreMesh(core_axis_name="c", subcore_axis_name="s")

    @functools.partial(
        pl.kernel,
        out_type=[
            jax.ShapeDtypeStruct((NC, n_acc, d), jnp.float32),
            jax.ShapeDtypeStruct((NC, n_acc, de), jnp.float32),
        ],
        mesh=mesh,
        compiler_params=pltpu.CompilerParams(use_tc_tiling_on_sc=False),
        scratch_types=[
            pltpu.VMEM_SHARED((n_acc, d), jnp.float32),
            pltpu.VMEM_SHARED((n_acc, de), jnp.float32),
            [pltpu.VMEM((CHUNK,), jnp.int32)] * 2,
            [pltpu.VMEM((CHUNK,), jnp.int32)] * 2,
            [pltpu.VMEM((CHUNK, d), jnp.float32)] * 2,
            [pltpu.VMEM((CHUNK, de), jnp.float32)] * 2,
            [pltpu.SemaphoreType.DMA] * 2,
            [pltpu.SemaphoreType.DMA] * 2,
            [pltpu.SemaphoreType.DMA] * 2,
        ],
    )
    def sc_kernel(xb_hbm, ei_hbm, ea_hbm, zx_hbm, zea_hbm, outx_hbm, outea_hbm,
                  accx, accea, i_v, j_v, rows_v, ea_v, ld, gt, sc):
        cid = lax.axis_index("c")
        sid = lax.axis_index("s")
        wid = sid * NC + cid

        pltpu.sync_copy(zx_hbm, accx.at[pl.ds(sid * rows_pt, rows_pt)])
        pltpu.sync_copy(zea_hbm, accea.at[pl.ds(sid * rows_pt, rows_pt)])
        plsc.subcore_barrier()

        base = wid * cpw

        def issue_loads(k, b):
            off = (base + k) * CHUNK
            pltpu.async_copy(ei_hbm.at[0, pl.ds(off, CHUNK)], i_v[b], ld[b])
            pltpu.async_copy(ei_hbm.at[1, pl.ds(off, CHUNK)], j_v[b], ld[b])
            pltpu.async_copy(ea_hbm.at[pl.ds(off, CHUNK), :], ea_v[b], ld[b])

        def wait_loads(k, b):
            off = (base + k) * CHUNK
            pltpu.make_async_copy(ei_hbm.at[0, pl.ds(off, CHUNK)], i_v[b], ld[b]).wait()
            pltpu.make_async_copy(ei_hbm.at[1, pl.ds(off, CHUNK)], j_v[b], ld[b]).wait()
            pltpu.make_async_copy(ea_hbm.at[pl.ds(off, CHUNK), :], ea_v[b], ld[b]).wait()

        def issue_gather(b):
            pltpu.async_copy(xb_hbm.at[j_v[b]], rows_v[b], gt[b])

        def wait_gather(b):
            pltpu.make_async_copy(xb_hbm.at[j_v[b]], rows_v[b], gt[b]).wait()

        def issue_scatter(b):
            pltpu.async_copy(rows_v[b], accx.at[i_v[b]], sc[b], add=True)
            pltpu.async_copy(ea_v[b], accea.at[i_v[b]], sc[b], add=True)

        def wait_scatter(b):
            pltpu.make_async_copy(rows_v[b], accx.at[i_v[b]], sc[b]).wait()
            pltpu.make_async_copy(ea_v[b], accea.at[i_v[b]], sc[b]).wait()

        def one(k, b, prefetch):
            nb = 1 - b
            wait_gather(b)
            issue_scatter(b)
            if prefetch:
                wait_scatter(nb)
                issue_loads(k + 1, nb)
                wait_loads(k + 1, nb)
                issue_gather(nb)

        issue_loads(0, 0)
        wait_loads(0, 0)
        issue_gather(0)
        issue_loads(1, 1)
        wait_gather(0)
        issue_scatter(0)
        wait_loads(1, 1)
        issue_gather(1)

        def pair(t, carry):
            k = 2 * t + 1
            one(k, 1, True)
            one(k + 1, 0, True)
            return carry

        lax.fori_loop(0, (cpw - 3) // 2, pair, 0)

        one(cpw - 2, 1, True)
        one(cpw - 1, 0, False)
        wait_scatter(1)
        wait_scatter(0)

        plsc.subcore_barrier()

        sl = pl.ds(sid * rows_pt, rows_pt)
        pltpu.sync_copy(accx.at[sl], outx_hbm.at[cid, sl])
        pltpu.sync_copy(accea.at[sl], outea_hbm.at[cid, sl])

    return sc_kernel(xb, ei, ea, zx, zea)


def _tc_dense(x, px, pea, eps, W1, b1, W2, b2, We, gamma, beta):
    n, d = x.shape
    de = We.shape[0]
    blk = 1000
    grid = n // blk

    epsv = jnp.reshape(1.0 + eps, (1, 1)).astype(jnp.float32)
    b1v = b1.reshape(1, d)
    b2v = b2.reshape(1, d)
    gammav = gamma.reshape(1, d)
    betav = beta.reshape(1, d)

    def body(x_ref, px_ref, pea_ref, eps_ref, w1_ref, b1_ref,
             w2_ref, b2_ref, we_ref, g_ref, bt_ref, o_ref):
        agg = px_ref[0] + px_ref[1]
        aea = pea_ref[0] + pea_ref[1]
        h = (eps_ref[0, 0] * x_ref[...] + agg
             + jnp.dot(aea, we_ref[...], preferred_element_type=jnp.float32))
        h = jnp.maximum(
            jnp.dot(h, w1_ref[...], preferred_element_type=jnp.float32)
            + b1_ref[...], 0.0)
        h = jnp.dot(h, w2_ref[...], preferred_element_type=jnp.float32) + b2_ref[...]
        mu = jnp.mean(h, axis=-1, keepdims=True)
        hc = h - mu
        var = jnp.mean(hc * hc, axis=-1, keepdims=True)
        o_ref[...] = hc * lax.rsqrt(var + 1e-5) * g_ref[...] + bt_ref[...]

    full = lambda i: (0, 0)
    return pl.pallas_call(
        body,
        grid=(grid,),
        in_specs=[
            pl.BlockSpec((blk, d), lambda i: (i, 0)),
            pl.BlockSpec((NC, blk, d), lambda i: (0, i, 0)),
            pl.BlockSpec((NC, blk, de), lambda i: (0, i, 0)),
            pl.BlockSpec((1, 1), full),
            pl.BlockSpec((d, d), full),
            pl.BlockSpec((1, d), full),
            pl.BlockSpec((d, d), full),
            pl.BlockSpec((1, d), full),
            pl.BlockSpec((de, d), full),
            pl.BlockSpec((1, d), full),
            pl.BlockSpec((1, d), full),
        ],
        out_specs=pl.BlockSpec((blk, d), lambda i: (i, 0)),
        out_shape=jax.ShapeDtypeStruct((n, d), jnp.float32),
    )(x, px, pea, epsv, W1, b1v, W2, b2v, We, gammav, betav)


def kernel(x, edge_index, edge_attr, eps, W1, b1, W2, b2, We, be, gamma, beta):
    n, d = x.shape
    e, de = edge_attr.shape

    ei = edge_index.astype(jnp.int32)
    xb = x + be.reshape(1, d)

    n_acc = -(-n // (NS * 8)) * (NS * 8)

    px, pea = _sc_segment_sums(xb, ei, edge_attr, n_acc)
    return _tc_dense(x, px, pea, eps, W1, b1, W2, b2, We, gamma, beta)


# speedup vs baseline: 4.9936x; 1.0022x over previous
import functools

import jax
import jax.numpy as jnp
from jax import lax
from jax.experimental import pallas as pl
from jax.experimental.pallas import tpu as pltpu
from jax.experimental.pallas import tpu_sc as plsc

CHUNK = 80
NC = 2
NS = 16
NW = NC * NS


def _sc_segment_sums(xb, ei1, ea_r, e, de, n_acc):
    n, d = xb.shape
    rows_pt = n_acc // NS
    assert e % (CHUNK * NW) == 0
    cpw = e // (CHUNK * NW)
    assert cpw >= 3 and cpw % 2 == 1

    zx = jnp.zeros((rows_pt, d), jnp.float32)
    zea = jnp.zeros((rows_pt, de), jnp.float32)

    mesh = plsc.VectorSubcoreMesh(core_axis_name="c", subcore_axis_name="s")

    @functools.partial(
        pl.kernel,
        out_type=[
            jax.ShapeDtypeStruct((NC, n_acc, d), jnp.float32),
            jax.ShapeDtypeStruct((NC, n_acc, de), jnp.float32),
        ],
        mesh=mesh,
        compiler_params=pltpu.CompilerParams(use_tc_tiling_on_sc=False),
        scratch_types=[
            pltpu.VMEM_SHARED((n_acc, d), jnp.float32),
            pltpu.VMEM_SHARED((n_acc, de), jnp.float32),
            [pltpu.VMEM((CHUNK,), jnp.int32)] * 2,
            [pltpu.VMEM((CHUNK,), jnp.int32)] * 2,
            [pltpu.VMEM((CHUNK, d), jnp.float32)] * 2,
            [pltpu.VMEM((CHUNK, de), jnp.float32)] * 2,
            [pltpu.SemaphoreType.DMA] * 2,
            [pltpu.SemaphoreType.DMA] * 2,
            [pltpu.SemaphoreType.DMA] * 2,
        ],
    )
    def sc_kernel(xb_hbm, ei_hbm, ea_hbm, zx_hbm, zea_hbm, outx_hbm, outea_hbm,
                  accx, accea, i_v, j_v, rows_v, ea_v, ld, gt, sc):
        cid = lax.axis_index("c")
        sid = lax.axis_index("s")
        wid = sid * NC + cid

        pltpu.sync_copy(zx_hbm, accx.at[pl.ds(sid * rows_pt, rows_pt)])
        pltpu.sync_copy(zea_hbm, accea.at[pl.ds(sid * rows_pt, rows_pt)])
        plsc.subcore_barrier()

        base = wid * cpw

        def issue_loads(k, b):
            off = (base + k) * CHUNK
            pltpu.async_copy(ei_hbm.at[pl.ds(off, CHUNK)], i_v[b], ld[b])
            pltpu.async_copy(ei_hbm.at[pl.ds(e + off, CHUNK)], j_v[b], ld[b])
            pltpu.async_copy(ea_hbm.at[base + k], ea_v[b], ld[b])

        def wait_loads(k, b):
            off = (base + k) * CHUNK
            pltpu.make_async_copy(ei_hbm.at[pl.ds(off, CHUNK)], i_v[b], ld[b]).wait()
            pltpu.make_async_copy(ei_hbm.at[pl.ds(e + off, CHUNK)], j_v[b], ld[b]).wait()
            pltpu.make_async_copy(ea_hbm.at[base + k], ea_v[b], ld[b]).wait()

        def issue_gather(b):
            pltpu.async_copy(xb_hbm.at[j_v[b]], rows_v[b], gt[b])

        def wait_gather(b):
            pltpu.make_async_copy(xb_hbm.at[j_v[b]], rows_v[b], gt[b]).wait()

        def issue_scatter(b):
            pltpu.async_copy(rows_v[b], accx.at[i_v[b]], sc[b], add=True)
            pltpu.async_copy(ea_v[b], accea.at[i_v[b]], sc[b], add=True)

        def wait_scatter(b):
            pltpu.make_async_copy(rows_v[b], accx.at[i_v[b]], sc[b]).wait()
            pltpu.make_async_copy(ea_v[b], accea.at[i_v[b]], sc[b]).wait()

        def one(k, b, prefetch):
            nb = 1 - b
            wait_gather(b)
            issue_scatter(b)
            if prefetch:
                wait_scatter(nb)
                issue_loads(k + 1, nb)
                wait_loads(k + 1, nb)
                issue_gather(nb)

        issue_loads(0, 0)
        wait_loads(0, 0)
        issue_gather(0)
        issue_loads(1, 1)
        wait_gather(0)
        issue_scatter(0)
        wait_loads(1, 1)
        issue_gather(1)

        def pair(t, carry):
            k = 2 * t + 1
            one(k, 1, True)
            one(k + 1, 0, True)
            return carry

        lax.fori_loop(0, (cpw - 3) // 2, pair, 0)

        one(cpw - 2, 1, True)
        one(cpw - 1, 0, False)
        wait_scatter(1)
        wait_scatter(0)

        plsc.subcore_barrier()

        sl = pl.ds(sid * rows_pt, rows_pt)
        pltpu.sync_copy(accx.at[sl], outx_hbm.at[cid, sl])
        pltpu.sync_copy(accea.at[sl], outea_hbm.at[cid, sl])

    return sc_kernel(xb, ei1, ea_r, zx, zea)


def _tc_dense(x, px, pea, eps, W1, b1, W2, b2, We, gamma, beta):
    n, d = x.shape
    de = We.shape[0]
    blk = 1000
    grid = n // blk

    epsv = jnp.reshape(1.0 + eps, (1, 1)).astype(jnp.float32)
    b1v = b1.reshape(1, d)
    b2v = b2.reshape(1, d)
    gammav = gamma.reshape(1, d)
    betav = beta.reshape(1, d)

    def body(x_ref, px_ref, pea_ref, eps_ref, w1_ref, b1_ref,
             w2_ref, b2_ref, we_ref, g_ref, bt_ref, o_ref):
        agg = px_ref[0] + px_ref[1]
        aea = pea_ref[0] + pea_ref[1]
        h = (eps_ref[0, 0] * x_ref[...] + agg
             + jnp.dot(aea, we_ref[...], preferred_element_type=jnp.float32))
        h = jnp.maximum(
            jnp.dot(h, w1_ref[...], preferred_element_type=jnp.float32)
            + b1_ref[...], 0.0)
        h = jnp.dot(h, w2_ref[...], preferred_element_type=jnp.float32) + b2_ref[...]
        mu = jnp.mean(h, axis=-1, keepdims=True)
        hc = h - mu
        var = jnp.mean(hc * hc, axis=-1, keepdims=True)
        o_ref[...] = hc * lax.rsqrt(var + 1e-5) * g_ref[...] + bt_ref[...]

    full = lambda i: (0, 0)
    return pl.pallas_call(
        body,
        grid=(grid,),
        in_specs=[
            pl.BlockSpec((blk, d), lambda i: (i, 0)),
            pl.BlockSpec((NC, blk, d), lambda i: (0, i, 0)),
            pl.BlockSpec((NC, blk, de), lambda i: (0, i, 0)),
            pl.BlockSpec((1, 1), full),
            pl.BlockSpec((d, d), full),
            pl.BlockSpec((1, d), full),
            pl.BlockSpec((d, d), full),
            pl.BlockSpec((1, d), full),
            pl.BlockSpec((de, d), full),
            pl.BlockSpec((1, d), full),
            pl.BlockSpec((1, d), full),
        ],
        out_specs=pl.BlockSpec((blk, d), lambda i: (i, 0)),
        out_shape=jax.ShapeDtypeStruct((n, d), jnp.float32),
    )(x, px, pea, epsv, W1, b1v, W2, b2v, We, gammav, betav)


def kernel(x, edge_index, edge_attr, eps, W1, b1, W2, b2, We, be, gamma, beta):
    n, d = x.shape
    e, de = edge_attr.shape

    ei1 = edge_index.astype(jnp.int32).reshape(2 * e)
    ea_r = edge_attr.reshape(e // CHUNK, CHUNK, de)
    xb = x + be.reshape(1, d)

    n_acc = -(-n // (NS * 8)) * (NS * 8)

    px, pea = _sc_segment_sums(xb, ei1, ea_r, e, de, n_acc)
    return _tc_dense(x, px, pea, eps, W1, b1, W2, b2, We, gamma, beta)


# speedup vs baseline: 6.0985x; 1.2213x over previous
import functools

import jax
import jax.numpy as jnp
from jax import lax
from jax.experimental import pallas as pl
from jax.experimental.pallas import tpu as pltpu
from jax.experimental.pallas import tpu_sc as plsc

CHUNK = 80
NC = 2
NS = 16
NW = NC * NS


def _sc_x_segment_sum(xb, ei1, n_acc):
    n, d = xb.shape
    e = ei1.shape[0] // 2
    rows_pt = n_acc // NS
    assert e % (CHUNK * NW) == 0
    cpw = e // (CHUNK * NW)
    assert cpw >= 5 and cpw % 2 == 1

    zx = jnp.zeros((rows_pt, d), jnp.float32)

    mesh = plsc.VectorSubcoreMesh(core_axis_name="c", subcore_axis_name="s")

    @functools.partial(
        pl.kernel,
        out_type=jax.ShapeDtypeStruct((NC, n_acc, d), jnp.float32),
        mesh=mesh,
        compiler_params=pltpu.CompilerParams(use_tc_tiling_on_sc=False),
        scratch_types=[
            pltpu.VMEM_SHARED((n_acc, d), jnp.float32),
            [pltpu.VMEM((CHUNK,), jnp.int32)] * 3,
            [pltpu.VMEM((CHUNK,), jnp.int32)] * 3,
            [pltpu.VMEM((CHUNK, d), jnp.float32)] * 2,
            [pltpu.SemaphoreType.DMA] * 3,
            [pltpu.SemaphoreType.DMA] * 2,
            [pltpu.SemaphoreType.DMA] * 2,
        ],
    )
    def sc_kernel(xb_hbm, ei_hbm, zx_hbm, outx_hbm,
                  accx, i_v, j_v, rows_v, ld, gt, sc):
        cid = lax.axis_index("c")
        sid = lax.axis_index("s")
        wid = sid * NC + cid

        pltpu.sync_copy(zx_hbm, accx.at[pl.ds(sid * rows_pt, rows_pt)])
        plsc.subcore_barrier()

        base = wid * cpw

        def issue_loads(k, b):
            off = (base + k) * CHUNK
            pltpu.async_copy(ei_hbm.at[pl.ds(off, CHUNK)], i_v[b], ld[b])
            pltpu.async_copy(ei_hbm.at[pl.ds(e + off, CHUNK)], j_v[b], ld[b])

        def wait_loads(b):
            pltpu.make_async_copy(ei_hbm.at[pl.ds(0, CHUNK)], i_v[b], ld[b]).wait()
            pltpu.make_async_copy(ei_hbm.at[pl.ds(0, CHUNK)], j_v[b], ld[b]).wait()

        def issue_gather(ib, rb):
            pltpu.async_copy(xb_hbm.at[j_v[ib]], rows_v[rb], gt[rb])

        def wait_gather(rb):
            pltpu.make_async_copy(xb_hbm.at[j_v[0]], rows_v[rb], gt[rb]).wait()

        def issue_scatter(ib, rb):
            pltpu.async_copy(rows_v[rb], accx.at[i_v[ib]], sc[rb], add=True)

        def wait_scatter(rb):
            pltpu.make_async_copy(rows_v[rb], accx.at[i_v[0]], sc[rb]).wait()

        def it_full(k, i3, r2):
            i3n = (i3 + 1) % 3
            i3nn = (i3 + 2) % 3
            r2n = 1 - r2
            wait_loads(i3n)
            wait_scatter(r2n)
            issue_gather(i3n, r2n)
            issue_loads(k + 2, i3nn)
            wait_gather(r2)
            issue_scatter(i3, r2)

        issue_loads(0, 0)
        issue_loads(1, 1)
        wait_loads(0)
        issue_gather(0, 0)
        wait_loads(1)
        issue_gather(1, 1)
        issue_loads(2, 2)
        wait_gather(0)
        issue_scatter(0, 0)
        wait_loads(2)
        wait_scatter(0)
        issue_gather(2, 0)
        issue_loads(3, 0)
        wait_gather(1)
        issue_scatter(1, 1)

        n6 = (cpw - 5) // 6

        def body6(t, carry):
            k0 = 2 + 6 * t
            for u in range(6):
                it_full(k0 + u, (2 + u) % 3, u % 2)
            return carry

        lax.fori_loop(0, n6, body6, 0)
        for u in range(6 * n6 + 2, cpw - 3):
            it_full(u, (u % 3), (u % 2))

        k = cpw - 3
        it_full(k, k % 3, k % 2)
        k = cpw - 2
        wait_loads((k + 1) % 3)
        wait_scatter(1 - (k % 2))
        issue_gather((k + 1) % 3, 1 - (k % 2))
        wait_gather(k % 2)
        issue_scatter(k % 3, k % 2)
        k = cpw - 1
        wait_gather(k % 2)
        issue_scatter(k % 3, k % 2)
        wait_scatter(0)
        wait_scatter(1)

        plsc.subcore_barrier()
        sl = pl.ds(sid * rows_pt, rows_pt)
        pltpu.sync_copy(accx.at[sl], outx_hbm.at[cid, sl])

    return sc_kernel(xb, ei1, zx)


def _sc_ea_segment_sum(ei1, ea_r, n_acc):
    e = ei1.shape[0] // 2
    de = ea_r.shape[2]
    rows_pt = n_acc // NS
    cpw = e // (CHUNK * NW)
    assert cpw >= 5 and cpw % 2 == 1

    zea = jnp.zeros((rows_pt, de), jnp.float32)

    mesh = plsc.VectorSubcoreMesh(core_axis_name="c", subcore_axis_name="s")

    @functools.partial(
        pl.kernel,
        out_type=jax.ShapeDtypeStruct((NC, n_acc, de), jnp.float32),
        mesh=mesh,
        compiler_params=pltpu.CompilerParams(use_tc_tiling_on_sc=False),
        scratch_types=[
            pltpu.VMEM_SHARED((n_acc, de), jnp.float32),
            [pltpu.VMEM((CHUNK,), jnp.int32)] * 3,
            [pltpu.VMEM((CHUNK, de), jnp.float32)] * 3,
            [pltpu.SemaphoreType.DMA] * 3,
            [pltpu.SemaphoreType.DMA] * 3,
        ],
    )
    def sc_kernel(ei_hbm, ea_hbm, zea_hbm, outea_hbm, accea, i_v, ea_v, ld, sc):
        cid = lax.axis_index("c")
        sid = lax.axis_index("s")
        wid = sid * NC + cid

        pltpu.sync_copy(zea_hbm, accea.at[pl.ds(sid * rows_pt, rows_pt)])
        plsc.subcore_barrier()

        base = wid * cpw

        def issue_loads(k, b):
            off = (base + k) * CHUNK
            pltpu.async_copy(ei_hbm.at[pl.ds(off, CHUNK)], i_v[b], ld[b])
            pltpu.async_copy(ea_hbm.at[base + k], ea_v[b], ld[b])

        def wait_loads(b):
            pltpu.make_async_copy(ei_hbm.at[pl.ds(0, CHUNK)], i_v[b], ld[b]).wait()
            pltpu.make_async_copy(ea_hbm.at[0], ea_v[b], ld[b]).wait()

        def issue_scatter(b):
            pltpu.async_copy(ea_v[b], accea.at[i_v[b]], sc[b], add=True)

        def wait_scatter(b):
            pltpu.make_async_copy(ea_v[b], accea.at[i_v[0]], sc[b]).wait()

        def it_full(k, b3):
            b3n = (b3 + 1) % 3
            b3nn = (b3 + 2) % 3
            wait_loads(b3)
            wait_scatter(b3nn)
            issue_scatter(b3)
            issue_loads(k + 2, b3nn)

        issue_loads(0, 0)
        issue_loads(1, 1)
        wait_loads(0)
        issue_scatter(0)
        issue_loads(2, 2)
        wait_loads(1)
        issue_scatter(1)
        wait_scatter(0)
        issue_loads(3, 0)

        n3 = (cpw - 4) // 3

        def body3(t, carry):
            k0 = 2 + 3 * t
            for u in range(3):
                it_full(k0 + u, (2 + u) % 3)
            return carry

        lax.fori_loop(0, n3, body3, 0)
        for u in range(3 * n3 + 2, cpw - 2):
            it_full(u, u % 3)

        for k in (cpw - 2, cpw - 1):
            wait_loads(k % 3)
            issue_scatter(k % 3)
        wait_scatter(0)
        wait_scatter(1)
        wait_scatter(2)

        plsc.subcore_barrier()
        sl = pl.ds(sid * rows_pt, rows_pt)
        pltpu.sync_copy(accea.at[sl], outea_hbm.at[cid, sl])

    return sc_kernel(ei1, ea_r, zea)


def _tc_dense(x, px, pea, eps, W1, b1, W2, b2, We, gamma, beta):
    n, d = x.shape
    de = We.shape[0]
    blk = 1000
    grid = n // blk

    epsv = jnp.reshape(1.0 + eps, (1, 1)).astype(jnp.float32)
    b1v = b1.reshape(1, d)
    b2v = b2.reshape(1, d)
    gammav = gamma.reshape(1, d)
    betav = beta.reshape(1, d)

    def body(x_ref, px_ref, pea_ref, eps_ref, w1_ref, b1_ref,
             w2_ref, b2_ref, we_ref, g_ref, bt_ref, o_ref):
        agg = px_ref[0] + px_ref[1]
        aea = pea_ref[0] + pea_ref[1]
        h = (eps_ref[0, 0] * x_ref[...] + agg
             + jnp.dot(aea, we_ref[...], preferred_element_type=jnp.float32))
        h = jnp.maximum(
            jnp.dot(h, w1_ref[...], preferred_element_type=jnp.float32)
            + b1_ref[...], 0.0)
        h = jnp.dot(h, w2_ref[...], preferred_element_type=jnp.float32) + b2_ref[...]
        mu = jnp.mean(h, axis=-1, keepdims=True)
        hc = h - mu
        var = jnp.mean(hc * hc, axis=-1, keepdims=True)
        o_ref[...] = hc * lax.rsqrt(var + 1e-5) * g_ref[...] + bt_ref[...]

    full = lambda i: (0, 0)
    return pl.pallas_call(
        body,
        grid=(grid,),
        in_specs=[
            pl.BlockSpec((blk, d), lambda i: (i, 0)),
            pl.BlockSpec((NC, blk, d), lambda i: (0, i, 0)),
            pl.BlockSpec((NC, blk, de), lambda i: (0, i, 0)),
            pl.BlockSpec((1, 1), full),
            pl.BlockSpec((d, d), full),
            pl.BlockSpec((1, d), full),
            pl.BlockSpec((d, d), full),
            pl.BlockSpec((1, d), full),
            pl.BlockSpec((de, d), full),
            pl.BlockSpec((1, d), full),
            pl.BlockSpec((1, d), full),
        ],
        out_specs=pl.BlockSpec((blk, d), lambda i: (i, 0)),
        out_shape=jax.ShapeDtypeStruct((n, d), jnp.float32),
    )(x, px, pea, epsv, W1, b1v, W2, b2v, We, gammav, betav)


def kernel(x, edge_index, edge_attr, eps, W1, b1, W2, b2, We, be, gamma, beta):
    n, d = x.shape
    e, de = edge_attr.shape

    ei1 = edge_index.astype(jnp.int32).reshape(2 * e)
    ea_r = edge_attr.reshape(e // CHUNK, CHUNK, de)
    xb = x + be.reshape(1, d)

    n_acc = -(-n // (NS * 8)) * (NS * 8)

    px = _sc_x_segment_sum(xb, ei1, n_acc)
    pea = _sc_ea_segment_sum(ei1, ea_r, n_acc)
    return _tc_dense(x, px, pea, eps, W1, b1, W2, b2, We, gamma, beta)
